# baseline (speedup 1.0000x reference)
import jax
import jax.numpy as jnp
from jax import lax
from jax.experimental import pallas as pl
from jax.experimental.pallas import tpu as pltpu
from jax.experimental.pallas import tpu_sc as plsc

_NUM_CORES = 2
_NUM_SUBCORES = 16
_NUM_WORKERS = _NUM_CORES * _NUM_SUBCORES
_LANES = 16

_VOCAB = 1000000
_BATCH = 4096
_MAXLEN = 200
_EMBED = 64
_BT = _BATCH // _NUM_WORKERS

_TB = 128
_NFULL = _VOCAB // _TB
_BLK_PER_W = -(-_NFULL // _NUM_WORKERS)


def _transpose_body(tokT_hbm, tail_hbm, out_hbm, staged, outbuf, sems):
    wid = lax.axis_index("s") * _NUM_CORES + lax.axis_index("c")

    iot = lax.iota(jnp.int32, _LANES)
    col_par = lax.bitwise_and(iot, 1) * _EMBED
    row_t = [lax.shift_right_logical(iot, 1) + 8 * t for t in range(_TB // _LANES)]

    def stage_start(b, k):
        off = pl.multiple_of(b * _TB, _TB)
        pltpu.async_copy(tokT_hbm.at[:, pl.ds(off, _TB)], staged.at[k], sems[k])

    def stage_wait(b, k):
        off = pl.multiple_of(b * _TB, _TB)
        pltpu.make_async_copy(
            tokT_hbm.at[:, pl.ds(off, _TB)], staged.at[k], sems[k]
        ).wait()

    def transpose(k):
        def erow(e, carry):
            col_idx = col_par + e
            for t in range(_TB // _LANES):
                v = staged[k, e, pl.ds(t * _LANES, _LANES)]
                plsc.store_scatter(outbuf.at[k], [row_t[t], col_idx], v)
            return carry

        lax.fori_loop(0, _EMBED, erow, 0, unroll=4)

    def wb_start(b, k):
        off = pl.multiple_of(b * (_TB // 2), _TB // 2)
        pltpu.async_copy(outbuf.at[k], out_hbm.at[pl.ds(off, _TB // 2)],
                         sems[2 + k])

    def wb_wait(b, k):
        off = pl.multiple_of(b * (_TB // 2), _TB // 2)
        pltpu.make_async_copy(
            outbuf.at[k], out_hbm.at[pl.ds(off, _TB // 2)], sems[2 + k]
        ).wait()

    def blk(g):
        return wid + g * _NUM_WORKERS

    stage_start(blk(0), 0)

    def pair(g2, carry):
        for kk in range(2):
            g = g2 * 2 + kk
            b = blk(g)

            @pl.when(b < _NFULL)
            def _():
                @pl.when(blk(g + 1) < _NFULL)
                def _():
                    stage_start(blk(g + 1), 1 - kk)
                stage_wait(b, kk)

                @pl.when(g >= 2)
                def _():
                    wb_wait(blk(g - 2), kk)
                transpose(kk)
                wb_start(b, kk)
        return carry

    lax.fori_loop(0, (_BLK_PER_W + 1) // 2, pair, 0)

    g_max = lax.div(_NFULL - 1 - wid, _NUM_WORKERS)
    for kk in range(2):
        g_k = g_max - lax.rem(g_max - kk + 2, 2)

        @pl.when(g_k >= 0)
        def _():
            wb_wait(blk(g_k), kk)

    @pl.when(wid == 0)
    def _():
        pltpu.sync_copy(tail_hbm, staged.at[0])
        transpose(0)
        pltpu.sync_copy(outbuf.at[0],
                        out_hbm.at[pl.ds((_VOCAB - _TB) // 2, _TB // 2)])


_NGB = 4
_NWB = 2


def _emb_body(xT_hbm, tok_hbm, pos_hbm, out_hbm, idxT, pos_v, rows, outT,
              sems):
    wid = lax.axis_index("s") * _NUM_CORES + lax.axis_index("c")

    pltpu.sync_copy(xT_hbm.at[:, pl.ds(wid * _BT, _BT)], idxT)
    pltpu.sync_copy(pos_hbm, pos_v)

    iot = lax.iota(jnp.int32, _LANES)
    f_lo = lax.bitwise_and(iot, 7)
    f_hi = [lax.shift_right_logical(iot, 3) + 2 * l
            for l in range(_EMBED // _LANES)]

    def g_start(m, k):
        pltpu.async_copy(tok_hbm.at[idxT.at[m]], rows.at[k], sems[k])

    def g_wait(m, k):
        pltpu.make_async_copy(tok_hbm.at[idxT.at[m]], rows.at[k],
                              sems[k]).wait()

    def trans_add(m, rk, ok):
        p = [pos_v[m, pl.ds(l * _LANES, _LANES)]
             for l in range(_EMBED // _LANES)]

        def tok_loop(t, carry):
            colv = jnp.full((_LANES,), 0, jnp.int32) + t
            for l in range(_EMBED // _LANES):
                v = rows[rk, t, pl.ds(l * _LANES, _LANES)] + p[l]
                plsc.store_scatter(outT.at[ok], [f_hi[l], f_lo, colv], v)
            return carry

        lax.fori_loop(0, _BT, tok_loop, 0, unroll=2)

    def wb_start(m, ok):
        pltpu.async_copy(outT.at[ok], out_hbm.at[m, :, wid], sems[_NGB + ok])

    def wb_wait(m, ok):
        pltpu.make_async_copy(outT.at[ok], out_hbm.at[m, :, wid],
                              sems[_NGB + ok]).wait()

    for k in range(_NGB):
        g_start(k, k)

    def quad(q, carry):
        for j in range(_NGB):
            m = q * _NGB + j
            ok = j % _NWB
            g_wait(m, j)

            @pl.when(m >= _NWB)
            def _():
                wb_wait(m - _NWB, ok)
            trans_add(m, j, ok)
            wb_start(m, ok)

            @pl.when(m + _NGB < _MAXLEN)
            def _():
                g_start(m + _NGB, j)
        return carry

    lax.fori_loop(0, _MAXLEN // _NGB, quad, 0)

    for ok in range(_NWB):
        wb_wait(_MAXLEN - _NWB + ok, ok)


def _mesh():
    return plsc.VectorSubcoreMesh(
        core_axis_name="c", subcore_axis_name="s",
        num_cores=_NUM_CORES, num_subcores=_NUM_SUBCORES,
    )


def _transpose(tokT, tail):
    return pl.kernel(
        _transpose_body,
        out_type=jax.ShapeDtypeStruct((_VOCAB // 2, 2 * _EMBED), jnp.float32),
        mesh=_mesh(),
        compiler_params=pltpu.CompilerParams(
            use_tc_tiling_on_sc=True, needs_layout_passes=False),
        scratch_types=[
            pltpu.VMEM((2, _EMBED, _TB), jnp.float32),
            pltpu.VMEM((2, _TB // 2, 2 * _EMBED), jnp.float32),
            [pltpu.SemaphoreType.DMA] * 4,
        ],
    )(tokT, tail)


def _emb(xT, tok_rm, pos_table):
    return pl.kernel(
        _emb_body,
        out_type=jax.ShapeDtypeStruct(
            (_MAXLEN, _EMBED // 8, _NUM_WORKERS, 8, _BT), jnp.float32),
        mesh=_mesh(),
        compiler_params=pltpu.CompilerParams(
            use_tc_tiling_on_sc=False, needs_layout_passes=False),
        scratch_types=[
            pltpu.VMEM((_MAXLEN, _BT), jnp.int32),
            pltpu.VMEM((_MAXLEN, _EMBED), jnp.float32),
            pltpu.VMEM((_NGB, _BT, _EMBED), jnp.float32),
            pltpu.VMEM((_NWB, _EMBED // 8, 8, _BT), jnp.float32),
            [pltpu.SemaphoreType.DMA] * (_NGB + _NWB),
        ],
    )(xT, tok_rm, pos_table)


@jax.jit
def _run(x, token_table, pos_table):
    tokT = token_table.T
    tail = token_table[_VOCAB - _TB:, :].T
    t128 = _transpose(tokT, tail)
    tok_rm = t128.reshape(_VOCAB, _EMBED)
    xT = x.T
    out5 = _emb(xT, tok_rm, pos_table)
    return out5.transpose(2, 4, 0, 1, 3).reshape(_BATCH, _MAXLEN, _EMBED)


def kernel(x, token_table, pos_table):
    return _run(jnp.asarray(x, jnp.int32), token_table, pos_table)


# speedup vs baseline: 2.0893x; 2.0893x over previous
import jax
import jax.numpy as jnp
from jax import lax
from jax.experimental import pallas as pl
from jax.experimental.pallas import tpu as pltpu
from jax.experimental.pallas import tpu_sc as plsc

_NUM_CORES = 2
_NUM_SUBCORES = 16
_NUM_WORKERS = _NUM_CORES * _NUM_SUBCORES
_LANES = 16

_VOCAB = 1000000
_BATCH = 4096
_MAXLEN = 200
_EMBED = 64
_BT = _BATCH // _NUM_WORKERS
_NE = _EMBED // _LANES

_NGB = 4
_NWB = 2
_SKEW = 2 * _EMBED + 1


def _emb_body(xT_hbm, tok_hbm, pos_hbm, out_hbm, idxT, pos_v, rows, skew,
              sems):
    wid = lax.axis_index("s") * _NUM_CORES + lax.axis_index("c")

    pltpu.sync_copy(xT_hbm.at[:, pl.ds(wid * _BT, _BT)], idxT)
    pltpu.sync_copy(pos_hbm, pos_v)

    iot = lax.iota(jnp.int32, _LANES)
    f_row = [iot + _LANES * l for l in range(_NE)]

    def g_start(m, k):
        pltpu.async_copy(tok_hbm.at[idxT.at[m]], rows.at[k], sems[k])

    def g_wait(m, k):
        pltpu.make_async_copy(tok_hbm.at[idxT.at[m]], rows.at[k],
                              sems[k]).wait()

    def trans_add(m, rk, ok):
        p = [pos_v[m, pl.ds(l * _LANES, _LANES)] for l in range(_NE)]

        def tok_loop(t, carry):
            colv = iot * 0 + t
            for l in range(_NE):
                v = rows[rk, t, pl.ds(l * _LANES, _LANES)] + p[l]
                plsc.store_scatter(skew.at[ok], [f_row[l], colv], v)
            return carry

        lax.fori_loop(0, _BT, tok_loop, 0, unroll=2)

    def wb_start(m, ok):
        for et in range(_EMBED // 8):
            pltpu.async_copy(skew.at[ok, pl.ds(8 * et, 8), pl.ds(0, _BT)],
                             out_hbm.at[m, et, wid], sems[_NGB + ok])

    def wb_wait(m, ok):
        for et in range(_EMBED // 8):
            pltpu.make_async_copy(
                skew.at[ok, pl.ds(8 * et, 8), pl.ds(0, _BT)],
                out_hbm.at[m, et, wid], sems[_NGB + ok]).wait()

    for k in range(_NGB):
        g_start(k, k)

    def quad(q, carry):
        for j in range(_NGB):
            m = q * _NGB + j
            ok = j % _NWB
            g_wait(m, j)

            @pl.when(m >= _NWB)
            def _():
                wb_wait(m - _NWB, ok)
            trans_add(m, j, ok)
            wb_start(m, ok)

            @pl.when(m + _NGB < _MAXLEN)
            def _():
                g_start(m + _NGB, j)
        return carry

    lax.fori_loop(0, _MAXLEN // _NGB, quad, 0)

    for ok in range(_NWB):
        wb_wait(_MAXLEN - _NWB + ok, ok)


def _mesh():
    return plsc.VectorSubcoreMesh(
        core_axis_name="c", subcore_axis_name="s",
        num_cores=_NUM_CORES, num_subcores=_NUM_SUBCORES,
    )


def _emb(xT, tok_rm, pos_table):
    return pl.kernel(
        _emb_body,
        out_type=jax.ShapeDtypeStruct(
            (_MAXLEN, _EMBED // 8, _NUM_WORKERS, 8, _BT), jnp.float32),
        mesh=_mesh(),
        compiler_params=pltpu.CompilerParams(
            use_tc_tiling_on_sc=False, needs_layout_passes=False),
        scratch_types=[
            pltpu.VMEM((_MAXLEN, _BT), jnp.int32),
            pltpu.VMEM((_MAXLEN, _EMBED), jnp.float32),
            pltpu.VMEM((_NGB, _BT, _EMBED), jnp.float32),
            pltpu.VMEM((_NWB, _EMBED, _SKEW), jnp.float32),
            [pltpu.SemaphoreType.DMA] * (_NGB + _NWB),
        ],
    )(xT, tok_rm, pos_table)


@jax.jit
def _run(x, token_table, pos_table):
    xT = x.T
    out5 = _emb(xT, token_table, pos_table)
    return out5.transpose(2, 4, 0, 1, 3).reshape(_BATCH, _MAXLEN, _EMBED)


def kernel(x, token_table, pos_table):
    return _run(jnp.asarray(x, jnp.int32), token_table, pos_table)
